# Initial kernel scaffold; baseline (speedup 1.0000x reference)
import jax
import jax.numpy as jnp
from jax.experimental import pallas as pl


def kernel(token_ids, word_ids, attention_mask, emb1, emb2, W1, b1, gamma, beta, Wf, bf):
    raise NotImplementedError("write your pallas kernel here")


# speedup vs baseline: 5.3953x; 5.3953x over previous
import functools

import jax
import jax.numpy as jnp
from jax.experimental import pallas as pl
from jax.experimental.pallas import tpu as pltpu
from jax.experimental.pallas import tpu_sc as plsc

_B, _L = 4096, 50
_R = _B * _L
_D = 64
_H = 256
_C = 1000
_EPS = 1e-5

_GW = 256
_BB = 64
_RB = _BB * _L
_NB = _B // _BB


_NW = 32
_PW = _R // _NW
_CH = 320
_NCH = _PW // _CH
_DW = 2 * _D


def _sc_gather(cat, idx1, idx2):
    mesh = plsc.VectorSubcoreMesh(core_axis_name="c", subcore_axis_name="s")

    @functools.partial(
        pl.kernel,
        out_type=(
            jax.ShapeDtypeStruct((_R, _DW), jnp.float32),
            jax.ShapeDtypeStruct((_R, _DW), jnp.float32),
        ),
        mesh=mesh,
        scratch_types=[
            pltpu.VMEM((_CH,), jnp.int32),
            pltpu.VMEM((_CH,), jnp.int32),
            pltpu.VMEM((_CH, _DW), jnp.float32),
            pltpu.VMEM((_CH, _DW), jnp.float32),
            pltpu.SemaphoreType.DMA,
            pltpu.SemaphoreType.DMA,
        ],
    )
    def k(cat_hbm, i1_hbm, i2_hbm, o1_hbm, o2_hbm,
          idx1_v, idx2_v, rows1_v, rows2_v, sem1, sem2):
        wid = jax.lax.axis_index("s") * 2 + jax.lax.axis_index("c")

        @pl.loop(0, _NCH)
        def _(c):
            base = wid * _PW + c * _CH
            pltpu.sync_copy(i1_hbm.at[pl.ds(base, _CH)], idx1_v)
            pltpu.sync_copy(i2_hbm.at[pl.ds(base, _CH)], idx2_v)
            cp1 = pltpu.async_copy(cat_hbm.at[idx1_v], rows1_v, sem1)
            cp2 = pltpu.async_copy(cat_hbm.at[idx2_v], rows2_v, sem2)
            cp1.wait()
            cp2.wait()
            pltpu.sync_copy(rows1_v, o1_hbm.at[pl.ds(base, _CH)])
            pltpu.sync_copy(rows2_v, o2_hbm.at[pl.ds(base, _CH)])

    return k(cat, idx1, idx2)


def _tc_body(x1_ref, x2_ref, w1a_ref, w1b_ref, b1_ref, gamma_ref, beta_ref,
             wft_ref, bf_ref, mask_ref, out_ref, acc_ref, sc_ref):
    p = pl.program_id(0)
    i = pl.program_id(1)

    @pl.when(jnp.logical_and(p == 0, i == 0))
    def _():
        acc_ref[...] = jnp.zeros_like(acc_ref)

    x1 = x1_ref[:, :_D]
    x2 = x2_ref[:, _D:]
    h = (jnp.dot(x1, w1a_ref[...], preferred_element_type=jnp.float32)
         + jnp.dot(x2, w1b_ref[...], preferred_element_type=jnp.float32)
         + b1_ref[...])

    @pl.when(p == 0)
    def _():
        acc_ref[0, :] += jnp.sum(h, axis=0)
        acc_ref[1, :] += jnp.sum(h * h, axis=0)

    @pl.when(p == 1)
    def _():
        @pl.when(i == 0)
        def _():
            inv_n = jnp.float32(1.0 / _R)
            mu = acc_ref[0, :] * inv_n
            var = acc_ref[1, :] * inv_n - mu * mu
            scale = gamma_ref[0, :] / jnp.sqrt(var + jnp.float32(_EPS))
            sc_ref[0, :] = scale
            sc_ref[1, :] = beta_ref[0, :] - mu * scale

        hn = jnp.maximum(h * sc_ref[0, :][None, :] + sc_ref[1, :][None, :], 0.0)
        pooled_sum = hn.reshape(_BB, _L, _H).sum(axis=1)
        denom = jnp.sum(mask_ref[...], axis=1, keepdims=True)
        pooled = pooled_sum / denom
        out_ref[...] = (jnp.dot(pooled, wft_ref[...],
                                preferred_element_type=jnp.float32)
                        + bf_ref[...])


def _tc_pipeline(x1, x2, w1a, w1b, b1, gamma, beta, wft, bf, mask):
    return pl.pallas_call(
        _tc_body,
        grid=(2, _NB),
        in_specs=[
            pl.BlockSpec((_RB, _DW), lambda p, i: (i, 0)),
            pl.BlockSpec((_RB, _DW), lambda p, i: (i, 0)),
            pl.BlockSpec((_D, _H), lambda p, i: (0, 0)),
            pl.BlockSpec((_D, _H), lambda p, i: (0, 0)),
            pl.BlockSpec((1, _H), lambda p, i: (0, 0)),
            pl.BlockSpec((1, _H), lambda p, i: (0, 0)),
            pl.BlockSpec((1, _H), lambda p, i: (0, 0)),
            pl.BlockSpec((_H, _C), lambda p, i: (0, 0)),
            pl.BlockSpec((1, _C), lambda p, i: (0, 0)),
            pl.BlockSpec((_BB, _L), lambda p, i: (i, 0)),
        ],
        out_specs=pl.BlockSpec((_BB, _C), lambda p, i: (p * i, 0)),
        out_shape=jax.ShapeDtypeStruct((_B, _C), jnp.float32),
        scratch_shapes=[
            pltpu.VMEM((2, _H), jnp.float32),
            pltpu.VMEM((2, _H), jnp.float32),
        ],
    )(x1, x2, w1a, w1b, b1, gamma, beta, wft, bf, mask)


def kernel(token_ids, word_ids, attention_mask, emb1, emb2, W1, b1, gamma,
           beta, Wf, bf):
    idx1 = token_ids.astype(jnp.int32).reshape(_R)
    idx2 = word_ids.astype(jnp.int32).reshape(_R)
    cat = jnp.concatenate([emb1, emb2], axis=1)
    x1, x2 = _sc_gather(cat, idx1, idx2)

    w1a = W1[:, :_D].T
    w1b = W1[:, _D:].T
    wft = Wf.T
    return _tc_pipeline(
        x1, x2, w1a, w1b,
        b1.reshape(1, _H), gamma.reshape(1, _H), beta.reshape(1, _H),
        wft, bf.reshape(1, _C), attention_mask,
    )


# speedup vs baseline: 6.1344x; 1.1370x over previous
import functools

import jax
import jax.numpy as jnp
from jax.experimental import pallas as pl
from jax.experimental.pallas import tpu as pltpu
from jax.experimental.pallas import tpu_sc as plsc

_B, _L = 4096, 50
_R = _B * _L
_D = 64
_H = 256
_C = 1000
_EPS = 1e-5

_BB = 64
_RB = _BB * _L
_NB = _B // _BB

_NW = 32
_PW = _R // _NW
_CH = 320
_NCH = _PW // _CH
_DW = 2 * _D


def _sc_gather(cat, idx1, idx2):
    mesh = plsc.VectorSubcoreMesh(core_axis_name="c", subcore_axis_name="s")

    @functools.partial(
        pl.kernel,
        out_type=(
            jax.ShapeDtypeStruct((_R, _DW), jnp.float32),
            jax.ShapeDtypeStruct((_R, _DW), jnp.float32),
        ),
        mesh=mesh,
        scratch_types=[
            pltpu.VMEM((_CH,), jnp.int32),
            pltpu.VMEM((_CH,), jnp.int32),
            pltpu.VMEM((_CH, _DW), jnp.float32),
            pltpu.VMEM((_CH, _DW), jnp.float32),
            pltpu.SemaphoreType.DMA,
            pltpu.SemaphoreType.DMA,
        ],
    )
    def k(cat_hbm, i1_hbm, i2_hbm, o1_hbm, o2_hbm,
          idx1_v, idx2_v, rows1_v, rows2_v, sem1, sem2):
        wid = jax.lax.axis_index("s") * 2 + jax.lax.axis_index("c")

        @pl.loop(0, _NCH)
        def _(c):
            base = wid * _PW + c * _CH
            pltpu.sync_copy(i1_hbm.at[pl.ds(base, _CH)], idx1_v)
            pltpu.sync_copy(i2_hbm.at[pl.ds(base, _CH)], idx2_v)
            cp1 = pltpu.async_copy(cat_hbm.at[idx1_v], rows1_v, sem1)
            cp2 = pltpu.async_copy(cat_hbm.at[idx2_v], rows2_v, sem2)
            cp1.wait()
            cp2.wait()
            pltpu.sync_copy(rows1_v, o1_hbm.at[pl.ds(base, _CH)])
            pltpu.sync_copy(rows2_v, o2_hbm.at[pl.ds(base, _CH)])

    return k(cat, idx1, idx2)


def _tc_a_body(x1_ref, x2_ref, w1_ref, b1_ref, xcb_ref, stats_ref, acc_ref):
    i = pl.program_id(0)

    @pl.when(i == 0)
    def _():
        acc_ref[...] = jnp.zeros_like(acc_ref)

    xcb = jnp.concatenate([x1_ref[:, :_D], x2_ref[:, _D:]],
                          axis=1).astype(jnp.bfloat16)
    xcb_ref[...] = xcb
    h = (jnp.dot(xcb, w1_ref[...], preferred_element_type=jnp.float32)
         + b1_ref[...])
    acc_ref[0, :] += jnp.sum(h, axis=0)
    acc_ref[1, :] += jnp.sum(h * h, axis=0)

    @pl.when(i == _NB - 1)
    def _():
        stats_ref[...] = acc_ref[...]


def _tc_a(x1, x2, w1, b1):
    return pl.pallas_call(
        _tc_a_body,
        grid=(_NB,),
        in_specs=[
            pl.BlockSpec((_RB, _DW), lambda i: (i, 0)),
            pl.BlockSpec((_RB, _DW), lambda i: (i, 0)),
            pl.BlockSpec((_DW, _H), lambda i: (0, 0)),
            pl.BlockSpec((1, _H), lambda i: (0, 0)),
        ],
        out_specs=[
            pl.BlockSpec((_RB, _DW), lambda i: (i, 0)),
            pl.BlockSpec((2, _H), lambda i: (0, 0)),
        ],
        out_shape=[
            jax.ShapeDtypeStruct((_R, _DW), jnp.bfloat16),
            jax.ShapeDtypeStruct((2, _H), jnp.float32),
        ],
        scratch_shapes=[pltpu.VMEM((2, _H), jnp.float32)],
    )(x1, x2, w1, b1)


def _tc_b_body(xcb_ref, stats_ref, w1_ref, b1_ref, gamma_ref, beta_ref,
               wft_ref, bf_ref, mask_ref, out_ref, sc_ref):
    i = pl.program_id(0)

    @pl.when(i == 0)
    def _():
        inv_n = jnp.float32(1.0 / _R)
        mu = stats_ref[0, :] * inv_n
        var = stats_ref[1, :] * inv_n - mu * mu
        scale = gamma_ref[0, :] / jnp.sqrt(var + jnp.float32(_EPS))
        sc_ref[0, :] = scale
        sc_ref[1, :] = beta_ref[0, :] - mu * scale

    h = (jnp.dot(xcb_ref[...], w1_ref[...], preferred_element_type=jnp.float32)
         + b1_ref[...])
    hn = jnp.maximum(h * sc_ref[0, :][None, :] + sc_ref[1, :][None, :], 0.0)
    pooled_sum = hn.reshape(_BB, _L, _H).sum(axis=1)
    denom = jnp.sum(mask_ref[...], axis=1, keepdims=True)
    pooled = (pooled_sum / denom).astype(jnp.bfloat16)
    out_ref[...] = (jnp.dot(pooled, wft_ref[...],
                            preferred_element_type=jnp.float32)
                    + bf_ref[...])


def _tc_b(xcb, stats, w1, b1, gamma, beta, wft, bf, mask):
    return pl.pallas_call(
        _tc_b_body,
        grid=(_NB,),
        in_specs=[
            pl.BlockSpec((_RB, _DW), lambda i: (i, 0)),
            pl.BlockSpec((2, _H), lambda i: (0, 0)),
            pl.BlockSpec((_DW, _H), lambda i: (0, 0)),
            pl.BlockSpec((1, _H), lambda i: (0, 0)),
            pl.BlockSpec((1, _H), lambda i: (0, 0)),
            pl.BlockSpec((1, _H), lambda i: (0, 0)),
            pl.BlockSpec((_H, _C), lambda i: (0, 0)),
            pl.BlockSpec((1, _C), lambda i: (0, 0)),
            pl.BlockSpec((_BB, _L), lambda i: (i, 0)),
        ],
        out_specs=pl.BlockSpec((_BB, _C), lambda i: (i, 0)),
        out_shape=jax.ShapeDtypeStruct((_B, _C), jnp.float32),
        scratch_shapes=[pltpu.VMEM((2, _H), jnp.float32)],
    )(xcb, stats, w1, b1, gamma, beta, wft, bf, mask)


def kernel(token_ids, word_ids, attention_mask, emb1, emb2, W1, b1, gamma,
           beta, Wf, bf):
    idx1 = token_ids.astype(jnp.int32).reshape(_R)
    idx2 = word_ids.astype(jnp.int32).reshape(_R)
    cat = jnp.concatenate([emb1, emb2], axis=1)
    x1, x2 = _sc_gather(cat, idx1, idx2)

    w1 = W1.T.astype(jnp.bfloat16)
    wft = Wf.T.astype(jnp.bfloat16)
    b1r = b1.reshape(1, _H)
    xcb, stats = _tc_a(x1, x2, w1, b1r)
    return _tc_b(
        xcb, stats, w1, b1r, gamma.reshape(1, _H), beta.reshape(1, _H),
        wft, bf.reshape(1, _C), attention_mask,
    )


# speedup vs baseline: 6.5569x; 1.0689x over previous
import functools

import jax
import jax.numpy as jnp
from jax.experimental import pallas as pl
from jax.experimental.pallas import tpu as pltpu
from jax.experimental.pallas import tpu_sc as plsc

_B, _L = 4096, 50
_R = _B * _L
_D = 64
_H = 256
_C = 1000
_EPS = 1e-5

_BB = 64
_RB = _BB * _L
_NB = _B // _BB

_NW = 32
_PW = _R // _NW
_CH = 400
_NCH = _PW // _CH
_DW = 2 * _D


def _sc_gather(cat, idx1, idx2):
    mesh = plsc.VectorSubcoreMesh(core_axis_name="c", subcore_axis_name="s")

    @functools.partial(
        pl.kernel,
        out_type=(
            jax.ShapeDtypeStruct((_R, _DW), jnp.float32),
            jax.ShapeDtypeStruct((_R, _DW), jnp.float32),
        ),
        mesh=mesh,
        scratch_types=[
            pltpu.VMEM((_CH,), jnp.int32),
            pltpu.VMEM((_CH,), jnp.int32),
            pltpu.VMEM((_CH, _DW), jnp.float32),
            pltpu.VMEM((_CH, _DW), jnp.float32),
            pltpu.SemaphoreType.DMA,
            pltpu.SemaphoreType.DMA,
        ],
    )
    def k(cat_hbm, i1_hbm, i2_hbm, o1_hbm, o2_hbm,
          idx1_v, idx2_v, rows1_v, rows2_v, sem1, sem2):
        wid = jax.lax.axis_index("s") * 2 + jax.lax.axis_index("c")

        @pl.loop(0, _NCH)
        def _(c):
            base = wid * _PW + c * _CH
            pltpu.sync_copy(i1_hbm.at[pl.ds(base, _CH)], idx1_v)
            pltpu.sync_copy(i2_hbm.at[pl.ds(base, _CH)], idx2_v)
            cp1 = pltpu.async_copy(cat_hbm.at[idx1_v], rows1_v, sem1)
            cp2 = pltpu.async_copy(cat_hbm.at[idx2_v], rows2_v, sem2)
            cp1.wait()
            cp2.wait()
            pltpu.sync_copy(rows1_v, o1_hbm.at[pl.ds(base, _CH)])
            pltpu.sync_copy(rows2_v, o2_hbm.at[pl.ds(base, _CH)])

    return k(cat, idx1, idx2)


def _tc_a_body(x1_ref, x2_ref, w1_ref, b1_ref, xcb_ref, stats_ref, acc_ref):
    i = pl.program_id(0)

    @pl.when(i == 0)
    def _():
        acc_ref[...] = jnp.zeros_like(acc_ref)

    xcb = jnp.concatenate([x1_ref[:, :_D], x2_ref[:, _D:]],
                          axis=1).astype(jnp.bfloat16)
    xcb_ref[...] = xcb
    h = (jnp.dot(xcb, w1_ref[...], preferred_element_type=jnp.float32)
         + b1_ref[...])
    acc_ref[0, :] += jnp.sum(h, axis=0)
    acc_ref[1, :] += jnp.sum(h * h, axis=0)

    @pl.when(i == _NB - 1)
    def _():
        stats_ref[...] = acc_ref[...]


def _tc_a(x1, x2, w1, b1):
    return pl.pallas_call(
        _tc_a_body,
        grid=(_NB,),
        in_specs=[
            pl.BlockSpec((_RB, _DW), lambda i: (i, 0)),
            pl.BlockSpec((_RB, _DW), lambda i: (i, 0)),
            pl.BlockSpec((_DW, _H), lambda i: (0, 0)),
            pl.BlockSpec((1, _H), lambda i: (0, 0)),
        ],
        out_specs=[
            pl.BlockSpec((_RB, _DW), lambda i: (i, 0)),
            pl.BlockSpec((2, _H), lambda i: (0, 0)),
        ],
        out_shape=[
            jax.ShapeDtypeStruct((_R, _DW), jnp.bfloat16),
            jax.ShapeDtypeStruct((2, _H), jnp.float32),
        ],
        scratch_shapes=[pltpu.VMEM((2, _H), jnp.float32)],
    )(x1, x2, w1, b1)


def _tc_b_body(xcb_ref, stats_ref, w1_ref, b1_ref, gamma_ref, beta_ref,
               wft_ref, bf_ref, mask_ref, pool_ref, out_ref,
               w1s_ref, bias_ref):
    i = pl.program_id(0)

    @pl.when(i == 0)
    def _():
        inv_n = jnp.float32(1.0 / _R)
        mu = stats_ref[0, :] * inv_n
        var = stats_ref[1, :] * inv_n - mu * mu
        scale = gamma_ref[0, :] / jnp.sqrt(var + jnp.float32(_EPS))
        w1s_ref[...] = (w1_ref[...].astype(jnp.float32)
                        * scale[None, :]).astype(jnp.bfloat16)
        bias_ref[0, :] = (b1_ref[0, :] - mu) * scale + beta_ref[0, :]

    h = (jnp.dot(xcb_ref[...], w1s_ref[...], preferred_element_type=jnp.float32)
         + bias_ref[0, :][None, :])
    hn = jnp.maximum(h, 0.0).astype(jnp.bfloat16)
    pooled_sum = jnp.dot(pool_ref[...], hn,
                         preferred_element_type=jnp.float32)
    denom = jnp.sum(mask_ref[...], axis=1, keepdims=True)
    pooled = (pooled_sum / denom).astype(jnp.bfloat16)
    out_ref[...] = (jnp.dot(pooled, wft_ref[...],
                            preferred_element_type=jnp.float32)
                    + bf_ref[...])


def _tc_b(xcb, stats, w1, b1, gamma, beta, wft, bf, mask, pool):
    return pl.pallas_call(
        _tc_b_body,
        grid=(_NB,),
        in_specs=[
            pl.BlockSpec((_RB, _DW), lambda i: (i, 0)),
            pl.BlockSpec((2, _H), lambda i: (0, 0)),
            pl.BlockSpec((_DW, _H), lambda i: (0, 0)),
            pl.BlockSpec((1, _H), lambda i: (0, 0)),
            pl.BlockSpec((1, _H), lambda i: (0, 0)),
            pl.BlockSpec((1, _H), lambda i: (0, 0)),
            pl.BlockSpec((_H, _C), lambda i: (0, 0)),
            pl.BlockSpec((1, _C), lambda i: (0, 0)),
            pl.BlockSpec((_BB, _L), lambda i: (i, 0)),
            pl.BlockSpec((_BB, _RB), lambda i: (0, 0)),
        ],
        out_specs=pl.BlockSpec((_BB, _C), lambda i: (i, 0)),
        out_shape=jax.ShapeDtypeStruct((_B, _C), jnp.float32),
        scratch_shapes=[
            pltpu.VMEM((_DW, _H), jnp.bfloat16),
            pltpu.VMEM((1, _H), jnp.float32),
        ],
    )(xcb, stats, w1, b1, gamma, beta, wft, bf, mask, pool)


def kernel(token_ids, word_ids, attention_mask, emb1, emb2, W1, b1, gamma,
           beta, Wf, bf):
    idx1 = token_ids.astype(jnp.int32).reshape(_R)
    idx2 = word_ids.astype(jnp.int32).reshape(_R)
    cat = jnp.concatenate([emb1, emb2], axis=1)
    x1, x2 = _sc_gather(cat, idx1, idx2)

    w1 = W1.T.astype(jnp.bfloat16)
    wft = Wf.T.astype(jnp.bfloat16)
    b1r = b1.reshape(1, _H)
    pool = (jnp.arange(_RB, dtype=jnp.int32)[None, :] // _L
            == jnp.arange(_BB, dtype=jnp.int32)[:, None]).astype(jnp.bfloat16)
    xcb, stats = _tc_a(x1, x2, w1, b1r)
    return _tc_b(
        xcb, stats, w1, b1r, gamma.reshape(1, _H), beta.reshape(1, _H),
        wft, bf.reshape(1, _C), attention_mask, pool,
    )
